# Initial kernel scaffold; baseline (speedup 1.0000x reference)
import jax
import jax.numpy as jnp
from jax.experimental import pallas as pl


def kernel(x, edge_index, edge_weight, batch, W0, b0, g0, bt0, W1, b1, g1, bt1, W2, b2, g2, bt2):
    raise NotImplementedError("write your pallas kernel here")


# speedup vs baseline: 2.6508x; 2.6508x over previous
import functools

import jax
import jax.numpy as jnp
from jax import lax
from jax.experimental import pallas as pl
from jax.experimental.pallas import tpu as pltpu

N = 10000
E = 320000
D_IN = 128
H = 64
B = 8
EPS = 1e-5
SCALE = 1.0 / (1.0 + EPS) ** 0.5


def _dense_pre_body(h_ref, w_ref, dinv_ref, h2_ref, p_ref):
    h2 = jnp.dot(h_ref[...], w_ref[...], preferred_element_type=jnp.float32)
    h2_ref[...] = h2
    p_ref[...] = h2 * dinv_ref[...]


def _dense_pre(h, w, dinv):
    return pl.pallas_call(
        _dense_pre_body,
        out_shape=(
            jax.ShapeDtypeStruct((N, w.shape[1]), jnp.float32),
            jax.ShapeDtypeStruct((N, w.shape[1]), jnp.float32),
        ),
    )(h, w, dinv)


def _dense_post_body(q_ref, h2_ref, dinv_ref, b_ref, gbt_ref, out_ref):
    dinv = dinv_ref[...]
    acc = dinv * q_ref[...] + (dinv * dinv) * h2_ref[...] + b_ref[...]
    g = gbt_ref[0:1, :]
    bt = gbt_ref[1:2, :]
    out_ref[...] = jnp.maximum(acc * (SCALE * g) + bt, 0.0)


def _dense_post(q, h2, dinv, b, g, bt):
    gbt = jnp.stack([g, bt], axis=0)
    return pl.pallas_call(
        _dense_post_body,
        out_shape=jax.ShapeDtypeStruct((N, H), jnp.float32),
    )(q, h2, dinv, b.reshape(1, H), gbt)


def _pool_body(h_ref, batch_ref, out_ref):
    h = h_ref[...]
    seg = batch_ref[...]
    ids = lax.broadcasted_iota(jnp.int32, (B, N), 0)
    onehot = jnp.where(seg == ids, 1.0, 0.0)
    sums = jnp.dot(onehot, h, preferred_element_type=jnp.float32)
    counts = jnp.sum(onehot, axis=1, keepdims=True)
    mean = sums / jnp.maximum(counts, 1.0)
    out_ref[0:B, :] = mean
    out_ref[B : 2 * B, :] = sums


def _pool(h, batch):
    out = pl.pallas_call(
        _pool_body,
        out_shape=jax.ShapeDtypeStruct((2 * B, H), jnp.float32),
    )(h, batch.reshape(1, N).astype(jnp.int32))
    return jnp.concatenate([out[0:B], out[B : 2 * B]], axis=-1)


def kernel(x, edge_index, edge_weight, batch, W0, b0, g0, bt0, W1, b1, g1, bt1, W2, b2, g2, bt2):
    src = edge_index[0]
    dst = edge_index[1]
    ew = jnp.abs(edge_weight)
    deg = jnp.zeros((N,), jnp.float32).at[dst].add(ew) + 1.0
    dinv = lax.rsqrt(deg)
    dinv_col = dinv[:, None]

    h = x
    for (W, b, g, bt) in [(W0, b0, g0, bt0), (W1, b1, g1, bt1), (W2, b2, g2, bt2)]:
        h2, p = _dense_pre(h, W, dinv_col)
        msg = p[src] * ew[:, None]
        q = jnp.zeros((N, H), jnp.float32).at[dst].add(msg)
        h = _dense_post(q, h2, dinv_col, b, g, bt)
    return _pool(h, batch)


# speedup vs baseline: 6.5059x; 2.4544x over previous
import functools

import jax
import jax.numpy as jnp
from jax import lax
from jax.experimental import pallas as pl
from jax.experimental.pallas import tpu as pltpu
from jax.experimental.pallas import tpu_sc as plsc

N = 10000
E = 320000
D_IN = 128
H = 64
B = 8
EPS = 1e-5
SCALE = 1.0 / (1.0 + EPS) ** 0.5

NC = 2
NS = 16
NW = NC * NS
CH = 128
NCHUNK = 80
EPW = NCHUNK * CH
EPAD = NW * EPW
NPAD = 10240
NPS = NPAD // NS

_sc_mesh = plsc.VectorSubcoreMesh(core_axis_name="c", subcore_axis_name="s",
                                  num_cores=NC, num_subcores=NS)


def _mp_body(p_hbm, src_hbm, dst_hbm, ew_hbm, qout_hbm,
             q_sh, idx_s, idx_d, wv, rows, zbuf, sem, sem2):
    c = lax.axis_index("c")
    s = lax.axis_index("s")
    w = c * NS + s

    zeros16 = jnp.zeros((16,), jnp.float32)
    def _zrow(i, _):
        for k in range(H // 16):
            zbuf[i, pl.ds(16 * k, 16)] = zeros16
        return 0
    lax.fori_loop(0, CH, _zrow, 0)
    for rep in range(NPS // CH):
        pltpu.sync_copy(zbuf, q_sh.at[pl.ds(s * NPS + rep * CH, CH)])

    pltpu.sync_copy(src_hbm.at[pl.ds(w * NCHUNK, NCHUNK)], idx_s)
    pltpu.sync_copy(dst_hbm.at[pl.ds(w * NCHUNK, NCHUNK)], idx_d)
    pltpu.sync_copy(ew_hbm.at[pl.ds(w * NCHUNK, NCHUNK)], wv)

    plsc.subcore_barrier()

    def _chunk(j, _):
        pltpu.async_copy(p_hbm.at[idx_s.at[j]], rows, sem).wait()
        def _scale(m, _):
            wvec = wv[j, pl.ds(16 * m, 16)]
            for g in range(16):
                e = 16 * m + g
                we = wvec[g]
                for k in range(H // 16):
                    rows[e, pl.ds(16 * k, 16)] = rows[e, pl.ds(16 * k, 16)] * we
            return 0
        lax.fori_loop(0, CH // 16, _scale, 0)
        pltpu.async_copy(rows, q_sh.at[idx_d.at[j]], sem2, add=True).wait()
        return 0

    lax.fori_loop(0, NCHUNK, _chunk, 0)

    plsc.subcore_barrier()

    pltpu.sync_copy(q_sh.at[pl.ds(s * NPS, NPS)],
                    qout_hbm.at[c, pl.ds(s * NPS, NPS)])


@functools.partial(
    pl.kernel,
    out_type=jax.ShapeDtypeStruct((NC, NPAD, H), jnp.float32),
    mesh=_sc_mesh,
    scratch_types=[
        pltpu.VMEM_SHARED((NPAD, H), jnp.float32),
        pltpu.VMEM((NCHUNK, CH), jnp.int32),
        pltpu.VMEM((NCHUNK, CH), jnp.int32),
        pltpu.VMEM((NCHUNK, CH), jnp.float32),
        pltpu.VMEM((CH, H), jnp.float32),
        pltpu.VMEM((CH, H), jnp.float32),
        pltpu.SemaphoreType.DMA,
        pltpu.SemaphoreType.DMA,
    ],
    compiler_params=pltpu.CompilerParams(use_tc_tiling_on_sc=False),
)
def _message_pass(p_hbm, src_hbm, dst_hbm, ew_hbm, qout_hbm,
                  q_sh, idx_s, idx_d, wv, rows, zbuf, sem, sem2):
    _mp_body(p_hbm, src_hbm, dst_hbm, ew_hbm, qout_hbm,
             q_sh, idx_s, idx_d, wv, rows, zbuf, sem, sem2)


def _dense_pre_body(h_ref, w_ref, dinv_ref, h2_ref, p_ref):
    h2 = jnp.dot(h_ref[...], w_ref[...], preferred_element_type=jnp.float32)
    h2_ref[...] = h2
    p_ref[...] = h2 * dinv_ref[...]


def _dense_pre(h, w, dinv):
    return pl.pallas_call(
        _dense_pre_body,
        out_shape=(
            jax.ShapeDtypeStruct((N, w.shape[1]), jnp.float32),
            jax.ShapeDtypeStruct((N, w.shape[1]), jnp.float32),
        ),
    )(h, w, dinv)


def _dense_post_body(q_ref, h2_ref, dinv_ref, b_ref, gbt_ref, out_ref):
    q = q_ref[0, :N, :] + q_ref[1, :N, :]
    dinv = dinv_ref[...]
    acc = dinv * q + (dinv * dinv) * h2_ref[...] + b_ref[...]
    g = gbt_ref[0:1, :]
    bt = gbt_ref[1:2, :]
    out_ref[...] = jnp.maximum(acc * (SCALE * g) + bt, 0.0)


def _dense_post(qp, h2, dinv, b, g, bt):
    gbt = jnp.stack([g, bt], axis=0)
    return pl.pallas_call(
        _dense_post_body,
        out_shape=jax.ShapeDtypeStruct((N, H), jnp.float32),
    )(qp, h2, dinv, b.reshape(1, H), gbt)


def _pool_body(h_ref, batch_ref, out_ref):
    h = h_ref[...]
    seg = batch_ref[...]
    ids = lax.broadcasted_iota(jnp.int32, (B, N), 0)
    onehot = jnp.where(seg == ids, 1.0, 0.0)
    sums = jnp.dot(onehot, h, preferred_element_type=jnp.float32)
    counts = jnp.sum(onehot, axis=1, keepdims=True)
    mean = sums / jnp.maximum(counts, 1.0)
    out_ref[0:B, :] = mean
    out_ref[B : 2 * B, :] = sums


def _pool(h, batch):
    out = pl.pallas_call(
        _pool_body,
        out_shape=jax.ShapeDtypeStruct((2 * B, H), jnp.float32),
    )(h, batch.reshape(1, N).astype(jnp.int32))
    return jnp.concatenate([out[0:B], out[B : 2 * B]], axis=-1)


def kernel(x, edge_index, edge_weight, batch, W0, b0, g0, bt0, W1, b1, g1, bt1, W2, b2, g2, bt2):
    src = edge_index[0].astype(jnp.int32)
    dst = edge_index[1].astype(jnp.int32)
    ew = jnp.abs(edge_weight)
    deg = jnp.zeros((N,), jnp.float32).at[dst].add(ew) + 1.0
    dinv = lax.rsqrt(deg)
    dinv_col = dinv[:, None]

    pad = EPAD - E
    src2d = jnp.concatenate([src, jnp.zeros((pad,), jnp.int32)]).reshape(-1, CH)
    dst2d = jnp.concatenate([dst, jnp.zeros((pad,), jnp.int32)]).reshape(-1, CH)
    ew2d = jnp.concatenate([ew, jnp.zeros((pad,), jnp.float32)]).reshape(-1, CH)

    h = x
    for (W, b, g, bt) in [(W0, b0, g0, bt0), (W1, b1, g1, bt1), (W2, b2, g2, bt2)]:
        h2, p = _dense_pre(h, W, dinv_col)
        qp = _message_pass(p, src2d, dst2d, ew2d)
        h = _dense_post(qp, h2, dinv_col, b, g, bt)
    return _pool(h, batch)


# speedup vs baseline: 9.7224x; 1.4944x over previous
import functools

import jax
import jax.numpy as jnp
from jax import lax
from jax.experimental import pallas as pl
from jax.experimental.pallas import tpu as pltpu
from jax.experimental.pallas import tpu_sc as plsc

N = 10000
E = 320000
D_IN = 128
H = 64
B = 8
EPS = 1e-5
SCALE = 1.0 / (1.0 + EPS) ** 0.5

NC = 2
NS = 16
NW = NC * NS
CH = 128
NCHUNK = 80
EPW = NCHUNK * CH
EPAD = NW * EPW
NPAD = 10240
NPS = NPAD // NS

_sc_mesh = plsc.VectorSubcoreMesh(core_axis_name="c", subcore_axis_name="s",
                                  num_cores=NC, num_subcores=NS)


NB = 2


def _mp_body(p_hbm, src_hbm, dst_hbm, ew_hbm, qout_hbm,
             q_sh, idx_s, idx_d, wv, rows, sbuf, gsems, ssems):
    c = lax.axis_index("c")
    s = lax.axis_index("s")
    w = c * NS + s

    zeros16 = jnp.zeros((16,), jnp.float32)
    def _zrow(i, _):
        for k in range(H // 16):
            rows[0][i, pl.ds(16 * k, 16)] = zeros16
        return 0
    lax.fori_loop(0, CH, _zrow, 0)
    for rep in range(NPS // CH):
        pltpu.sync_copy(rows[0], q_sh.at[pl.ds(s * NPS + rep * CH, CH)])

    pltpu.sync_copy(src_hbm.at[pl.ds(w * NCHUNK, NCHUNK)], idx_s)
    pltpu.sync_copy(dst_hbm.at[pl.ds(w * NCHUNK, NCHUNK)], idx_d)
    pltpu.sync_copy(ew_hbm.at[pl.ds(w * NCHUNK, NCHUNK)], wv)

    plsc.subcore_barrier()

    def _scale(j, src, dst):
        def _grp(m, _):
            wvec = wv[j, pl.ds(16 * m, 16)]
            for g in range(16):
                e = 16 * m + g
                we = wvec[g]
                for k in range(H // 16):
                    dst[e, pl.ds(16 * k, 16)] = src[e, pl.ds(16 * k, 16)] * we
            return 0
        lax.fori_loop(0, CH // 16, _grp, 0)

    def _fire_gather(j, b):
        pltpu.async_copy(p_hbm.at[idx_s.at[j]], rows[b], gsems[b])

    def _wait_gather(j, b):
        pltpu.make_async_copy(p_hbm.at[idx_s.at[j]], rows[b], gsems[b]).wait()

    def _fire_scatter(j, b):
        pltpu.async_copy(sbuf[b], q_sh.at[idx_d.at[j]], ssems[b], add=True)

    def _wait_scatter(j, b):
        pltpu.make_async_copy(sbuf[b], q_sh.at[idx_d.at[j]], ssems[b]).wait()

    for b in range(NB):
        _fire_gather(b, b)

    def _super(t, _):
        j0 = t * NB
        for b in range(NB):
            j = j0 + b
            _wait_gather(j, b)

            @pl.when(t > 0)
            def _():
                _wait_scatter(j - NB, b)

            _scale(j, rows[b], sbuf[b])
            _fire_scatter(j, b)

            @pl.when(j + NB < NCHUNK)
            def _():
                _fire_gather(j + NB, b)
        return 0

    lax.fori_loop(0, NCHUNK // NB, _super, 0)
    for b in range(NB):
        _wait_scatter(NCHUNK - NB + b, b)

    plsc.subcore_barrier()

    pltpu.sync_copy(q_sh.at[pl.ds(s * NPS, NPS)],
                    qout_hbm.at[c, pl.ds(s * NPS, NPS)])


@functools.partial(
    pl.kernel,
    out_type=jax.ShapeDtypeStruct((NC, NPAD, H), jnp.float32),
    mesh=_sc_mesh,
    scratch_types=(
        [
            pltpu.VMEM_SHARED((NPAD, H), jnp.float32),
            pltpu.VMEM((NCHUNK, CH), jnp.int32),
            pltpu.VMEM((NCHUNK, CH), jnp.int32),
            pltpu.VMEM((NCHUNK, CH), jnp.float32),
        ]
        + [pltpu.VMEM((CH, H), jnp.float32)] * NB
        + [pltpu.VMEM((CH, H), jnp.float32)] * NB
        + [pltpu.SemaphoreType.DMA] * (2 * NB)
    ),
    compiler_params=pltpu.CompilerParams(use_tc_tiling_on_sc=False),
)
def _message_pass(p_hbm, src_hbm, dst_hbm, ew_hbm, qout_hbm,
                  q_sh, idx_s, idx_d, wv, *bufs):
    rows = list(bufs[0:NB])
    sbuf = list(bufs[NB : 2 * NB])
    gsems = list(bufs[2 * NB : 3 * NB])
    ssems = list(bufs[3 * NB : 4 * NB])
    _mp_body(p_hbm, src_hbm, dst_hbm, ew_hbm, qout_hbm,
             q_sh, idx_s, idx_d, wv, rows, sbuf, gsems, ssems)


def _dense_pre_body(h_ref, w_ref, dinv_ref, h2_ref, p_ref):
    h2 = jnp.dot(h_ref[...], w_ref[...], preferred_element_type=jnp.float32)
    h2_ref[...] = h2
    p_ref[...] = h2 * dinv_ref[...]


def _dense_pre(h, w, dinv):
    return pl.pallas_call(
        _dense_pre_body,
        out_shape=(
            jax.ShapeDtypeStruct((N, w.shape[1]), jnp.float32),
            jax.ShapeDtypeStruct((N, w.shape[1]), jnp.float32),
        ),
    )(h, w, dinv)


def _dense_post_body(q_ref, h2_ref, dinv_ref, b_ref, gbt_ref, out_ref):
    q = q_ref[0, :N, :] + q_ref[1, :N, :]
    dinv = dinv_ref[...]
    acc = dinv * q + (dinv * dinv) * h2_ref[...] + b_ref[...]
    g = gbt_ref[0:1, :]
    bt = gbt_ref[1:2, :]
    out_ref[...] = jnp.maximum(acc * (SCALE * g) + bt, 0.0)


def _dense_post(qp, h2, dinv, b, g, bt):
    gbt = jnp.stack([g, bt], axis=0)
    return pl.pallas_call(
        _dense_post_body,
        out_shape=jax.ShapeDtypeStruct((N, H), jnp.float32),
    )(qp, h2, dinv, b.reshape(1, H), gbt)


def _pool_body(h_ref, batch_ref, out_ref):
    h = h_ref[...]
    seg = batch_ref[...]
    ids = lax.broadcasted_iota(jnp.int32, (B, N), 0)
    onehot = jnp.where(seg == ids, 1.0, 0.0)
    sums = jnp.dot(onehot, h, preferred_element_type=jnp.float32)
    counts = jnp.sum(onehot, axis=1, keepdims=True)
    mean = sums / jnp.maximum(counts, 1.0)
    out_ref[0:B, :] = mean
    out_ref[B : 2 * B, :] = sums


def _pool(h, batch):
    out = pl.pallas_call(
        _pool_body,
        out_shape=jax.ShapeDtypeStruct((2 * B, H), jnp.float32),
    )(h, batch.reshape(1, N).astype(jnp.int32))
    return jnp.concatenate([out[0:B], out[B : 2 * B]], axis=-1)


def kernel(x, edge_index, edge_weight, batch, W0, b0, g0, bt0, W1, b1, g1, bt1, W2, b2, g2, bt2):
    src = edge_index[0].astype(jnp.int32)
    dst = edge_index[1].astype(jnp.int32)
    ew = jnp.abs(edge_weight)
    deg = jnp.zeros((N,), jnp.float32).at[dst].add(ew) + 1.0
    dinv = lax.rsqrt(deg)
    dinv_col = dinv[:, None]

    pad = EPAD - E
    src2d = jnp.concatenate([src, jnp.zeros((pad,), jnp.int32)]).reshape(-1, CH)
    dst2d = jnp.concatenate([dst, jnp.zeros((pad,), jnp.int32)]).reshape(-1, CH)
    ew2d = jnp.concatenate([ew, jnp.zeros((pad,), jnp.float32)]).reshape(-1, CH)

    h = x
    for (W, b, g, bt) in [(W0, b0, g0, bt0), (W1, b1, g1, bt1), (W2, b2, g2, bt2)]:
        h2, p = _dense_pre(h, W, dinv_col)
        qp = _message_pass(p, src2d, dst2d, ew2d)
        h = _dense_post(qp, h2, dinv_col, b, g, bt)
    return _pool(h, batch)


# speedup vs baseline: 10.6039x; 1.0907x over previous
import functools

import jax
import jax.numpy as jnp
from jax import lax
from jax.experimental import pallas as pl
from jax.experimental.pallas import tpu as pltpu
from jax.experimental.pallas import tpu_sc as plsc

N = 10000
E = 320000
D_IN = 128
H = 64
B = 8
EPS = 1e-5
SCALE = 1.0 / (1.0 + EPS) ** 0.5

NC = 2
NS = 16
NW = NC * NS
CH = 128
NCHUNK = 80
EPW = NCHUNK * CH
EPAD = NW * EPW
NPAD = 10000
NPS = NPAD // NS
ZCH = 125

_sc_mesh = plsc.VectorSubcoreMesh(core_axis_name="c", subcore_axis_name="s",
                                  num_cores=NC, num_subcores=NS)


NB = 2


def _mp_body(p_hbm, src_hbm, dst_hbm, ew_hbm, qout_hbm,
             q_sh, p_sh, idx_s, idx_d, wv, rows, gsems, ssems):
    c = lax.axis_index("c")
    s = lax.axis_index("s")
    w = c * NS + s

    zeros16 = jnp.zeros((16,), jnp.float32)
    def _zrow(i, _):
        for k in range(H // 16):
            rows[0][i, pl.ds(16 * k, 16)] = zeros16
        return 0
    lax.fori_loop(0, ZCH, _zrow, 0)
    for rep in range(NPS // ZCH):
        pltpu.sync_copy(rows[0].at[pl.ds(0, ZCH)],
                        q_sh.at[pl.ds(s * NPS + rep * ZCH, ZCH)])

    pltpu.sync_copy(p_hbm.at[pl.ds(s * NPS, NPS)], p_sh.at[pl.ds(s * NPS, NPS)])
    pltpu.sync_copy(src_hbm.at[pl.ds(w * NCHUNK, NCHUNK)], idx_s)
    pltpu.sync_copy(dst_hbm.at[pl.ds(w * NCHUNK, NCHUNK)], idx_d)
    pltpu.sync_copy(ew_hbm.at[pl.ds(w * NCHUNK, NCHUNK)], wv)

    plsc.subcore_barrier()

    def _scale(j, buf):
        def _grp(m, _):
            wvec = wv[j, pl.ds(16 * m, 16)]
            for g in range(16):
                e = 16 * m + g
                we = wvec[g]
                for k in range(H // 16):
                    buf[e, pl.ds(16 * k, 16)] = buf[e, pl.ds(16 * k, 16)] * we
            return 0
        lax.fori_loop(0, CH // 16, _grp, 0)

    def _fire_gather(j, b):
        pltpu.async_copy(p_sh.at[idx_s.at[j]], rows[b], gsems[b])

    def _wait_gather(j, b):
        pltpu.make_async_copy(p_sh.at[idx_s.at[j]], rows[b], gsems[b]).wait()

    def _fire_scatter(j, b):
        pltpu.async_copy(rows[b], q_sh.at[idx_d.at[j]], ssems[b], add=True)

    def _wait_scatter(j, b):
        pltpu.make_async_copy(rows[b], q_sh.at[idx_d.at[j]], ssems[b]).wait()

    _fire_gather(0, 0)

    def _step(j, _):
        for b in range(NB):
            jj = j * NB + b
            nb = 1 - b

            @pl.when(jj + 1 < NCHUNK)
            def _():
                @pl.when(jj >= 1)
                def _():
                    _wait_scatter(jj - 1, nb)
                _fire_gather(jj + 1, nb)

            _wait_gather(jj, b)
            _scale(jj, rows[b])
            _fire_scatter(jj, b)
        return 0

    lax.fori_loop(0, NCHUNK // NB, _step, 0)
    _wait_scatter(NCHUNK - 2, 0)
    _wait_scatter(NCHUNK - 1, 1)

    plsc.subcore_barrier()

    pltpu.sync_copy(q_sh.at[pl.ds(s * NPS, NPS)],
                    qout_hbm.at[c, pl.ds(s * NPS, NPS)])


@functools.partial(
    pl.kernel,
    out_type=jax.ShapeDtypeStruct((NC, NPAD, H), jnp.float32),
    mesh=_sc_mesh,
    scratch_types=(
        [
            pltpu.VMEM_SHARED((NPAD, H), jnp.float32),
            pltpu.VMEM_SHARED((NPAD, H), jnp.float32),
            pltpu.VMEM((NCHUNK, CH), jnp.int32),
            pltpu.VMEM((NCHUNK, CH), jnp.int32),
            pltpu.VMEM((NCHUNK, CH), jnp.float32),
        ]
        + [pltpu.VMEM((CH, H), jnp.float32)] * NB
        + [pltpu.SemaphoreType.DMA] * (2 * NB)
    ),
    compiler_params=pltpu.CompilerParams(use_tc_tiling_on_sc=False),
)
def _message_pass(p_hbm, src_hbm, dst_hbm, ew_hbm, qout_hbm,
                  q_sh, p_sh, idx_s, idx_d, wv, *bufs):
    rows = list(bufs[0:NB])
    gsems = list(bufs[NB : 2 * NB])
    ssems = list(bufs[2 * NB : 3 * NB])
    _mp_body(p_hbm, src_hbm, dst_hbm, ew_hbm, qout_hbm,
             q_sh, p_sh, idx_s, idx_d, wv, rows, gsems, ssems)


def _dense_pre_body(h_ref, w_ref, dinv_ref, h2_ref, p_ref):
    h2 = jnp.dot(h_ref[...], w_ref[...], preferred_element_type=jnp.float32)
    h2_ref[...] = h2
    p_ref[...] = h2 * dinv_ref[...]


def _dense_pre(h, w, dinv):
    return pl.pallas_call(
        _dense_pre_body,
        out_shape=(
            jax.ShapeDtypeStruct((N, w.shape[1]), jnp.float32),
            jax.ShapeDtypeStruct((N, w.shape[1]), jnp.float32),
        ),
    )(h, w, dinv)


def _dense_post_body(q_ref, h2_ref, dinv_ref, b_ref, gbt_ref, out_ref):
    q = q_ref[0, :N, :] + q_ref[1, :N, :]
    dinv = dinv_ref[...]
    acc = dinv * q + (dinv * dinv) * h2_ref[...] + b_ref[...]
    g = gbt_ref[0:1, :]
    bt = gbt_ref[1:2, :]
    out_ref[...] = jnp.maximum(acc * (SCALE * g) + bt, 0.0)


def _dense_post(qp, h2, dinv, b, g, bt):
    gbt = jnp.stack([g, bt], axis=0)
    return pl.pallas_call(
        _dense_post_body,
        out_shape=jax.ShapeDtypeStruct((N, H), jnp.float32),
    )(qp, h2, dinv, b.reshape(1, H), gbt)


def _pool_body(h_ref, batch_ref, out_ref):
    h = h_ref[...]
    seg = batch_ref[...]
    ids = lax.broadcasted_iota(jnp.int32, (B, N), 0)
    onehot = jnp.where(seg == ids, 1.0, 0.0)
    sums = jnp.dot(onehot, h, preferred_element_type=jnp.float32)
    counts = jnp.sum(onehot, axis=1, keepdims=True)
    mean = sums / jnp.maximum(counts, 1.0)
    out_ref[0:B, :] = mean
    out_ref[B : 2 * B, :] = sums


def _pool(h, batch):
    out = pl.pallas_call(
        _pool_body,
        out_shape=jax.ShapeDtypeStruct((2 * B, H), jnp.float32),
    )(h, batch.reshape(1, N).astype(jnp.int32))
    return jnp.concatenate([out[0:B], out[B : 2 * B]], axis=-1)


def kernel(x, edge_index, edge_weight, batch, W0, b0, g0, bt0, W1, b1, g1, bt1, W2, b2, g2, bt2):
    src = edge_index[0].astype(jnp.int32)
    dst = edge_index[1].astype(jnp.int32)
    ew = jnp.abs(edge_weight)
    deg = jnp.zeros((N,), jnp.float32).at[dst].add(ew) + 1.0
    dinv = lax.rsqrt(deg)
    dinv_col = dinv[:, None]

    pad = EPAD - E
    src2d = jnp.concatenate([src, jnp.zeros((pad,), jnp.int32)]).reshape(-1, CH)
    dst2d = jnp.concatenate([dst, jnp.zeros((pad,), jnp.int32)]).reshape(-1, CH)
    ew2d = jnp.concatenate([ew, jnp.zeros((pad,), jnp.float32)]).reshape(-1, CH)

    h = x
    for (W, b, g, bt) in [(W0, b0, g0, bt0), (W1, b1, g1, bt1), (W2, b2, g2, bt2)]:
        h2, p = _dense_pre(h, W, dinv_col)
        qp = _message_pass(p, src2d, dst2d, ew2d)
        h = _dense_post(qp, h2, dinv_col, b, g, bt)
    return _pool(h, batch)


# speedup vs baseline: 11.4751x; 1.0822x over previous
import functools

import jax
import jax.numpy as jnp
from jax import lax
from jax.experimental import pallas as pl
from jax.experimental.pallas import tpu as pltpu
from jax.experimental.pallas import tpu_sc as plsc

N = 10000
E = 320000
D_IN = 128
H = 64
B = 8
EPS = 1e-5
SCALE = 1.0 / (1.0 + EPS) ** 0.5

NC = 2
NS = 16
NW = NC * NS
CH = 128
NCHUNK = 80
EPW = NCHUNK * CH
EPAD = NW * EPW
NPAD = 10000
NPS = NPAD // NS
ZCH = 125

_sc_mesh = plsc.VectorSubcoreMesh(core_axis_name="c", subcore_axis_name="s",
                                  num_cores=NC, num_subcores=NS)


NB = 2


def _mp_body(p_hbm, src_hbm, dst_hbm, ew_hbm, qout_hbm,
             q_sh, idx_s, idx_d, wv, rows, sbuf, gsems, ssems):
    c = lax.axis_index("c")
    s = lax.axis_index("s")
    w = c * NS + s

    zeros16 = jnp.zeros((16,), jnp.float32)
    def _zrow(i, _):
        for k in range(H // 16):
            sbuf[0][i, pl.ds(16 * k, 16)] = zeros16
        return 0
    lax.fori_loop(0, ZCH, _zrow, 0)
    for rep in range(NPS // ZCH):
        pltpu.sync_copy(sbuf[0].at[pl.ds(0, ZCH)],
                        q_sh.at[pl.ds(s * NPS + rep * ZCH, ZCH)])

    pltpu.sync_copy(src_hbm.at[pl.ds(w * NCHUNK, NCHUNK)], idx_s)
    pltpu.sync_copy(dst_hbm.at[pl.ds(w * NCHUNK, NCHUNK)], idx_d)
    pltpu.sync_copy(ew_hbm.at[pl.ds(w * NCHUNK, NCHUNK)], wv)

    plsc.subcore_barrier()

    def _scale(j, src, dst):
        def _grp(m, _):
            wvec = wv[j, pl.ds(16 * m, 16)]
            for g in range(16):
                e = 16 * m + g
                we = wvec[g]
                for k in range(H // 32):
                    x = src[e, pl.ds(32 * k, 32)]
                    a, bb = plsc.unpack(x, format=plsc.PackFormat.INTERLEAVED)
                    dst[e, pl.ds(32 * k, 16)] = a * we
                    dst[e, pl.ds(32 * k + 16, 16)] = bb * we
            return 0
        lax.fori_loop(0, CH // 16, _grp, 0)

    def _fire_gather(j, b):
        pltpu.async_copy(p_hbm.at[idx_s.at[j]], rows[b], gsems[b])

    def _wait_gather(j, b):
        pltpu.make_async_copy(p_hbm.at[idx_s.at[j]], rows[b], gsems[b]).wait()

    def _fire_scatter(j, b):
        pltpu.async_copy(sbuf[b], q_sh.at[idx_d.at[j]], ssems[b], add=True)

    def _wait_scatter(j, b):
        pltpu.make_async_copy(sbuf[b], q_sh.at[idx_d.at[j]], ssems[b]).wait()

    for b in range(NB):
        _fire_gather(b, b)

    def _super(t, _):
        j0 = t * NB
        for b in range(NB):
            j = j0 + b
            _wait_gather(j, b)

            @pl.when(t > 0)
            def _():
                _wait_scatter(j - NB, b)

            _scale(j, rows[b], sbuf[b])
            _fire_scatter(j, b)

            @pl.when(j + NB < NCHUNK)
            def _():
                _fire_gather(j + NB, b)
        return 0

    lax.fori_loop(0, NCHUNK // NB, _super, 0)
    for b in range(NB):
        _wait_scatter(NCHUNK - NB + b, b)

    plsc.subcore_barrier()

    pltpu.sync_copy(q_sh.at[pl.ds(s * NPS, NPS)],
                    qout_hbm.at[c, pl.ds(s * NPS, NPS)])


@functools.partial(
    pl.kernel,
    out_type=jax.ShapeDtypeStruct((NC, NPAD, H), jnp.float32),
    mesh=_sc_mesh,
    scratch_types=(
        [
            pltpu.VMEM_SHARED((NPAD, H), jnp.float32),
            pltpu.VMEM((NCHUNK, CH), jnp.int32),
            pltpu.VMEM((NCHUNK, CH), jnp.int32),
            pltpu.VMEM((NCHUNK, CH), jnp.float32),
        ]
        + [pltpu.VMEM((CH, H), jnp.bfloat16)] * NB
        + [pltpu.VMEM((CH, H), jnp.float32)] * NB
        + [pltpu.SemaphoreType.DMA] * (2 * NB)
    ),
    compiler_params=pltpu.CompilerParams(use_tc_tiling_on_sc=False,
                                        needs_layout_passes=False),
)
def _message_pass(p_hbm, src_hbm, dst_hbm, ew_hbm, qout_hbm,
                  q_sh, idx_s, idx_d, wv, *bufs):
    rows = list(bufs[0:NB])
    sbuf = list(bufs[NB : 2 * NB])
    gsems = list(bufs[2 * NB : 3 * NB])
    ssems = list(bufs[3 * NB : 4 * NB])
    _mp_body(p_hbm, src_hbm, dst_hbm, ew_hbm, qout_hbm,
             q_sh, idx_s, idx_d, wv, rows, sbuf, gsems, ssems)


def _dense_pre_body(h_ref, w_ref, dinv_ref, h2_ref, p_ref):
    h2 = jnp.dot(h_ref[...], w_ref[...], preferred_element_type=jnp.float32)
    h2_ref[...] = h2
    p_ref[...] = h2 * dinv_ref[...]


def _dense_pre(h, w, dinv):
    return pl.pallas_call(
        _dense_pre_body,
        out_shape=(
            jax.ShapeDtypeStruct((N, w.shape[1]), jnp.float32),
            jax.ShapeDtypeStruct((N, w.shape[1]), jnp.float32),
        ),
    )(h, w, dinv)


def _dense_post_body(q_ref, h2_ref, dinv_ref, b_ref, gbt_ref, out_ref):
    q = q_ref[0, :N, :] + q_ref[1, :N, :]
    dinv = dinv_ref[...]
    acc = dinv * q + (dinv * dinv) * h2_ref[...] + b_ref[...]
    g = gbt_ref[0:1, :]
    bt = gbt_ref[1:2, :]
    out_ref[...] = jnp.maximum(acc * (SCALE * g) + bt, 0.0)


def _dense_post(qp, h2, dinv, b, g, bt):
    gbt = jnp.stack([g, bt], axis=0)
    return pl.pallas_call(
        _dense_post_body,
        out_shape=jax.ShapeDtypeStruct((N, H), jnp.float32),
    )(qp, h2, dinv, b.reshape(1, H), gbt)


def _pool_body(h_ref, batch_ref, out_ref):
    h = h_ref[...]
    seg = batch_ref[...]
    ids = lax.broadcasted_iota(jnp.int32, (B, N), 0)
    onehot = jnp.where(seg == ids, 1.0, 0.0)
    sums = jnp.dot(onehot, h, preferred_element_type=jnp.float32)
    counts = jnp.sum(onehot, axis=1, keepdims=True)
    mean = sums / jnp.maximum(counts, 1.0)
    out_ref[0:B, :] = mean
    out_ref[B : 2 * B, :] = sums


def _pool(h, batch):
    out = pl.pallas_call(
        _pool_body,
        out_shape=jax.ShapeDtypeStruct((2 * B, H), jnp.float32),
    )(h, batch.reshape(1, N).astype(jnp.int32))
    return jnp.concatenate([out[0:B], out[B : 2 * B]], axis=-1)


def kernel(x, edge_index, edge_weight, batch, W0, b0, g0, bt0, W1, b1, g1, bt1, W2, b2, g2, bt2):
    src = edge_index[0].astype(jnp.int32)
    dst = edge_index[1].astype(jnp.int32)
    ew = jnp.abs(edge_weight)
    deg = jnp.zeros((N,), jnp.float32).at[dst].add(ew) + 1.0
    dinv = lax.rsqrt(deg)
    dinv_col = dinv[:, None]

    pad = EPAD - E
    src2d = jnp.concatenate([src, jnp.zeros((pad,), jnp.int32)]).reshape(-1, CH)
    dst2d = jnp.concatenate([dst, jnp.zeros((pad,), jnp.int32)]).reshape(-1, CH)
    ew2d = jnp.concatenate([ew, jnp.zeros((pad,), jnp.float32)]).reshape(-1, CH)

    posf = []
    for f in range(H):
        m, r = divmod(f, 32)
        posf.append(32 * m + (r // 2 if r % 2 == 0 else 16 + r // 2))
    posf = jnp.asarray(posf, jnp.int32)

    h = x
    for (W, b, g, bt) in [(W0, b0, g0, bt0), (W1, b1, g1, bt1), (W2, b2, g2, bt2)]:
        h2, p = _dense_pre(h, W, dinv_col)
        qp = _message_pass(p.astype(jnp.bfloat16), src2d, dst2d, ew2d)
        qp = jnp.take(qp, posf, axis=2)
        h = _dense_post(qp, h2, dinv_col, b, g, bt)
    return _pool(h, batch)


# speedup vs baseline: 18.7869x; 1.6372x over previous
import functools

import jax
import jax.numpy as jnp
import numpy as np
from jax import lax
from jax.experimental import pallas as pl
from jax.experimental.pallas import tpu as pltpu
from jax.experimental.pallas import tpu_sc as plsc

N = 10000
E = 320000
D_IN = 128
H = 64
B = 8
EPS = 1e-5
SCALE = 1.0 / (1.0 + EPS) ** 0.5

NC = 2
NS = 16
NW = NC * NS
CH = 128
NCHUNK = 80
EPAD = NW * NCHUNK * CH
NPAD = 10000
NPS = NPAD // NS
ZCH = 125
ND = 10240
NDS = ND // NS
NB = 2

_POSF = np.empty((H,), np.int32)
for _f in range(H):
    _m, _r = divmod(_f, 32)
    _POSF[_f] = 32 * _m + (_r // 2 if _r % 2 == 0 else 16 + _r // 2)
_PERM = (_POSF[None, :] == np.arange(H)[:, None]).astype(np.float32)

_sc_mesh = plsc.VectorSubcoreMesh(core_axis_name="c", subcore_axis_name="s",
                                  num_cores=NC, num_subcores=NS)
_sc_params = pltpu.CompilerParams(use_tc_tiling_on_sc=False,
                                  needs_layout_passes=False)


@functools.partial(
    pl.kernel,
    out_type=jax.ShapeDtypeStruct((NC, ND), jnp.float32),
    mesh=_sc_mesh,
    scratch_types=[
        pltpu.VMEM_SHARED((ND,), jnp.float32),
        pltpu.VMEM((NCHUNK, CH), jnp.int32),
        pltpu.VMEM((NCHUNK, CH), jnp.float32),
        pltpu.VMEM((CH,), jnp.float32),
        pltpu.SemaphoreType.DMA,
    ],
    compiler_params=_sc_params,
)
def _degree(dst_hbm, ew_hbm, degp_hbm, qd_sh, idx_d, wv, zb, sem):
    c = lax.axis_index("c")
    s = lax.axis_index("s")
    w = c * NS + s

    zeros16 = jnp.zeros((16,), jnp.float32)
    for k in range(CH // 16):
        zb[pl.ds(16 * k, 16)] = zeros16
    for rep in range(NDS // CH):
        pltpu.sync_copy(zb, qd_sh.at[pl.ds(s * NDS + rep * CH, CH)])

    pltpu.sync_copy(dst_hbm.at[pl.ds(w * NCHUNK, NCHUNK)], idx_d)
    pltpu.sync_copy(ew_hbm.at[pl.ds(w * NCHUNK, NCHUNK)], wv)

    plsc.subcore_barrier()

    def _fire(j, _):
        pltpu.async_copy(wv.at[j], qd_sh.at[idx_d.at[j]], sem, add=True)
        return 0
    lax.fori_loop(0, NCHUNK, _fire, 0)

    def _drain(j, _):
        pltpu.make_async_copy(wv.at[0], qd_sh.at[idx_d.at[0]], sem).wait()
        return 0
    lax.fori_loop(0, NCHUNK, _drain, 0)

    plsc.subcore_barrier()

    pltpu.sync_copy(qd_sh.at[pl.ds(s * NDS, NDS)],
                    degp_hbm.at[c, pl.ds(s * NDS, NDS)])


def _mp_body(p_hbm, src_hbm, dst_hbm, ew_hbm, qout_hbm,
             q_sh, idx_s, idx_d, wv, rows, sbuf, gsems, ssems):
    c = lax.axis_index("c")
    s = lax.axis_index("s")
    w = c * NS + s

    zeros16 = jnp.zeros((16,), jnp.float32)
    def _zrow(i, _):
        for k in range(H // 16):
            sbuf[0][i, pl.ds(16 * k, 16)] = zeros16
        return 0
    lax.fori_loop(0, ZCH, _zrow, 0)
    for rep in range(NPS // ZCH):
        pltpu.sync_copy(sbuf[0].at[pl.ds(0, ZCH)],
                        q_sh.at[pl.ds(s * NPS + rep * ZCH, ZCH)])

    pltpu.sync_copy(src_hbm.at[pl.ds(w * NCHUNK, NCHUNK)], idx_s)
    pltpu.sync_copy(dst_hbm.at[pl.ds(w * NCHUNK, NCHUNK)], idx_d)
    pltpu.sync_copy(ew_hbm.at[pl.ds(w * NCHUNK, NCHUNK)], wv)

    plsc.subcore_barrier()

    def _scale(j, src, dst):
        def _grp(m, _):
            wvec = wv[j, pl.ds(16 * m, 16)]
            for g in range(16):
                e = 16 * m + g
                we = wvec[g]
                for k in range(H // 32):
                    x = src[e, pl.ds(32 * k, 32)]
                    a, bb = plsc.unpack(x, format=plsc.PackFormat.INTERLEAVED)
                    dst[e, pl.ds(32 * k, 16)] = a * we
                    dst[e, pl.ds(32 * k + 16, 16)] = bb * we
            return 0
        lax.fori_loop(0, CH // 16, _grp, 0)

    def _fire_gather(j, b):
        pltpu.async_copy(p_hbm.at[idx_s.at[j]], rows[b], gsems[b])

    def _wait_gather(j, b):
        pltpu.make_async_copy(p_hbm.at[idx_s.at[j]], rows[b], gsems[b]).wait()

    def _fire_scatter(j, b):
        pltpu.async_copy(sbuf[b], q_sh.at[idx_d.at[j]], ssems[b], add=True)

    def _wait_scatter(j, b):
        pltpu.make_async_copy(sbuf[b], q_sh.at[idx_d.at[j]], ssems[b]).wait()

    for b in range(NB):
        _fire_gather(b, b)

    def _super(t, _):
        j0 = t * NB
        for b in range(NB):
            j = j0 + b
            _wait_gather(j, b)

            @pl.when(t > 0)
            def _():
                _wait_scatter(j - NB, b)

            _scale(j, rows[b], sbuf[b])
            _fire_scatter(j, b)

            @pl.when(j + NB < NCHUNK)
            def _():
                _fire_gather(j + NB, b)
        return 0

    lax.fori_loop(0, NCHUNK // NB, _super, 0)
    for b in range(NB):
        _wait_scatter(NCHUNK - NB + b, b)

    plsc.subcore_barrier()

    pltpu.sync_copy(q_sh.at[pl.ds(s * NPS, NPS)],
                    qout_hbm.at[c, pl.ds(s * NPS, NPS)])


@functools.partial(
    pl.kernel,
    out_type=jax.ShapeDtypeStruct((NC, NPAD, H), jnp.float32),
    mesh=_sc_mesh,
    scratch_types=(
        [
            pltpu.VMEM_SHARED((NPAD, H), jnp.float32),
            pltpu.VMEM((NCHUNK, CH), jnp.int32),
            pltpu.VMEM((NCHUNK, CH), jnp.int32),
            pltpu.VMEM((NCHUNK, CH), jnp.float32),
        ]
        + [pltpu.VMEM((CH, H), jnp.bfloat16)] * NB
        + [pltpu.VMEM((CH, H), jnp.float32)] * NB
        + [pltpu.SemaphoreType.DMA] * (2 * NB)
    ),
    compiler_params=_sc_params,
)
def _message_pass(p_hbm, src_hbm, dst_hbm, ew_hbm, qout_hbm,
                  q_sh, idx_s, idx_d, wv, *bufs):
    rows = list(bufs[0:NB])
    sbuf = list(bufs[NB : 2 * NB])
    gsems = list(bufs[2 * NB : 3 * NB])
    ssems = list(bufs[3 * NB : 4 * NB])
    _mp_body(p_hbm, src_hbm, dst_hbm, ew_hbm, qout_hbm,
             q_sh, idx_s, idx_d, wv, rows, sbuf, gsems, ssems)


def _tc_first_body(x_ref, w_ref, degp_ref, h2_ref, p_ref, dinv_ref):
    deg = degp_ref[0, 0:N] + degp_ref[1, 0:N] + 1.0
    dinv = lax.rsqrt(deg)[:, None]
    dinv_ref[...] = dinv
    h2 = jnp.dot(x_ref[...], w_ref[...], preferred_element_type=jnp.float32)
    h2_ref[...] = h2
    p_ref[...] = (h2 * dinv).astype(jnp.bfloat16)


def _tc_first(x, w0, degp):
    return pl.pallas_call(
        _tc_first_body,
        out_shape=(
            jax.ShapeDtypeStruct((N, H), jnp.float32),
            jax.ShapeDtypeStruct((N, H), jnp.bfloat16),
            jax.ShapeDtypeStruct((N, 1), jnp.float32),
        ),
    )(x, w0, degp)


def _tc_mid_body(q_ref, h2_ref, dinv_ref, bgbt_ref, perm_ref, wn_ref,
                 h2n_ref, pn_ref):
    qperm = q_ref[0, :N, :] + q_ref[1, :N, :]
    q = jnp.dot(qperm, perm_ref[...], preferred_element_type=jnp.float32)
    dinv = dinv_ref[...]
    b = bgbt_ref[0:1, :]
    g = bgbt_ref[1:2, :]
    bt = bgbt_ref[2:3, :]
    acc = dinv * q + (dinv * dinv) * h2_ref[...] + b
    h = jnp.maximum(acc * (SCALE * g) + bt, 0.0)
    h2n = jnp.dot(h, wn_ref[...], preferred_element_type=jnp.float32)
    h2n_ref[...] = h2n
    pn_ref[...] = (h2n * dinv).astype(jnp.bfloat16)


def _tc_mid(qp, h2, dinv, b, g, bt, perm, wn):
    bgbt = jnp.stack([b, g, bt], axis=0)
    return pl.pallas_call(
        _tc_mid_body,
        out_shape=(
            jax.ShapeDtypeStruct((N, H), jnp.float32),
            jax.ShapeDtypeStruct((N, H), jnp.bfloat16),
        ),
    )(qp, h2, dinv, bgbt, perm, wn)


def _tc_last_body(q_ref, h2_ref, dinv_ref, bgbt_ref, perm_ref, batch_ref,
                  out_ref):
    qperm = q_ref[0, :N, :] + q_ref[1, :N, :]
    q = jnp.dot(qperm, perm_ref[...], preferred_element_type=jnp.float32)
    dinv = dinv_ref[...]
    b = bgbt_ref[0:1, :]
    g = bgbt_ref[1:2, :]
    bt = bgbt_ref[2:3, :]
    acc = dinv * q + (dinv * dinv) * h2_ref[...] + b
    h = jnp.maximum(acc * (SCALE * g) + bt, 0.0)
    seg = batch_ref[...]
    ids = lax.broadcasted_iota(jnp.int32, (B, N), 0)
    onehot = jnp.where(seg == ids, 1.0, 0.0)
    sums = jnp.dot(onehot, h, preferred_element_type=jnp.float32)
    counts = jnp.sum(onehot, axis=1, keepdims=True)
    mean = sums / jnp.maximum(counts, 1.0)
    out_ref[0:B, :] = mean
    out_ref[B : 2 * B, :] = sums


def _tc_last(qp, h2, dinv, b, g, bt, perm, batch):
    bgbt = jnp.stack([b, g, bt], axis=0)
    return pl.pallas_call(
        _tc_last_body,
        out_shape=jax.ShapeDtypeStruct((2 * B, H), jnp.float32),
    )(qp, h2, dinv, bgbt, perm, batch.reshape(1, N).astype(jnp.int32))


def kernel(x, edge_index, edge_weight, batch, W0, b0, g0, bt0, W1, b1, g1, bt1, W2, b2, g2, bt2):
    src = edge_index[0].astype(jnp.int32)
    dst = edge_index[1].astype(jnp.int32)
    ew = jnp.abs(edge_weight)

    pad = EPAD - E
    src2d = jnp.concatenate([src, jnp.zeros((pad,), jnp.int32)]).reshape(-1, CH)
    dst2d = jnp.concatenate([dst, jnp.zeros((pad,), jnp.int32)]).reshape(-1, CH)
    ew2d = jnp.concatenate([ew, jnp.zeros((pad,), jnp.float32)]).reshape(-1, CH)

    perm = jnp.asarray(_PERM)

    degp = _degree(dst2d, ew2d)
    h2, p, dinv = _tc_first(x, W0, degp)
    qp = _message_pass(p, src2d, dst2d, ew2d)
    h2, p = _tc_mid(qp, h2, dinv, b0, g0, bt0, perm, W1)
    qp = _message_pass(p, src2d, dst2d, ew2d)
    h2, p = _tc_mid(qp, h2, dinv, b1, g1, bt1, perm, W2)
    qp = _message_pass(p, src2d, dst2d, ew2d)
    out = _tc_last(qp, h2, dinv, b2, g2, bt2, perm, batch)
    return jnp.concatenate([out[0:B], out[B : 2 * B]], axis=-1)


# speedup vs baseline: 20.4838x; 1.0903x over previous
import functools

import jax
import jax.numpy as jnp
import numpy as np
from jax import lax
from jax.experimental import pallas as pl
from jax.experimental.pallas import tpu as pltpu
from jax.experimental.pallas import tpu_sc as plsc

N = 10000
E = 320000
D_IN = 128
H = 64
B = 8
EPS = 1e-5
SCALE = 1.0 / (1.0 + EPS) ** 0.5

NC = 2
NS = 16
NW = NC * NS
CH = 128
NCHUNK = 80
EPAD = NW * NCHUNK * CH
NPAD = 10000
NPS = NPAD // NS
ZCH = 125
ND = 10240
NDS = ND // NS
NB = 2

_POSF = np.empty((H,), np.int32)
for _f in range(H):
    _m, _r = divmod(_f, 32)
    _POSF[_f] = 32 * _m + (_r // 2 if _r % 2 == 0 else 16 + _r // 2)
_PERM = (_POSF[None, :] == np.arange(H)[:, None]).astype(np.float32)

_sc_mesh = plsc.VectorSubcoreMesh(core_axis_name="c", subcore_axis_name="s",
                                  num_cores=NC, num_subcores=NS)
_sc_params = pltpu.CompilerParams(use_tc_tiling_on_sc=False,
                                  needs_layout_passes=False)


@functools.partial(
    pl.kernel,
    out_type=jax.ShapeDtypeStruct((NC, ND), jnp.float32),
    mesh=_sc_mesh,
    scratch_types=[
        pltpu.VMEM_SHARED((ND,), jnp.float32),
        pltpu.VMEM((NCHUNK, CH), jnp.int32),
        pltpu.VMEM((NCHUNK, CH), jnp.float32),
        pltpu.VMEM((CH,), jnp.float32),
        pltpu.SemaphoreType.DMA,
    ],
    compiler_params=_sc_params,
)
def _degree(dst_hbm, ew_hbm, degp_hbm, qd_sh, idx_d, wv, zb, sem):
    c = lax.axis_index("c")
    s = lax.axis_index("s")
    w = c * NS + s

    zeros16 = jnp.zeros((16,), jnp.float32)
    for k in range(CH // 16):
        zb[pl.ds(16 * k, 16)] = zeros16
    for rep in range(NDS // CH):
        pltpu.sync_copy(zb, qd_sh.at[pl.ds(s * NDS + rep * CH, CH)])

    pltpu.sync_copy(dst_hbm.at[pl.ds(w * NCHUNK, NCHUNK)], idx_d)
    pltpu.sync_copy(ew_hbm.at[pl.ds(w * NCHUNK, NCHUNK)], wv)

    plsc.subcore_barrier()

    def _fire(j, _):
        pltpu.async_copy(wv.at[j], qd_sh.at[idx_d.at[j]], sem, add=True)
        return 0
    lax.fori_loop(0, NCHUNK, _fire, 0)

    def _drain(j, _):
        pltpu.make_async_copy(wv.at[0], qd_sh.at[idx_d.at[0]], sem).wait()
        return 0
    lax.fori_loop(0, NCHUNK, _drain, 0)

    plsc.subcore_barrier()

    pltpu.sync_copy(qd_sh.at[pl.ds(s * NDS, NDS)],
                    degp_hbm.at[c, pl.ds(s * NDS, NDS)])


def _mp_body(p_hbm, src_hbm, dst_hbm, ew_hbm, qout_hbm,
             q_sh, idx_s, idx_d, wv, rows, sbuf, gsems, ssems):
    c = lax.axis_index("c")
    s = lax.axis_index("s")
    w = c * NS + s

    zeros16 = jnp.zeros((16,), jnp.float32)
    def _zrow(i, _):
        for k in range(H // 16):
            sbuf[0][i, pl.ds(16 * k, 16)] = zeros16
        return 0
    lax.fori_loop(0, ZCH, _zrow, 0)
    for rep in range(NPS // ZCH):
        pltpu.sync_copy(sbuf[0].at[pl.ds(0, ZCH)],
                        q_sh.at[pl.ds(s * NPS + rep * ZCH, ZCH)])

    pltpu.sync_copy(src_hbm.at[pl.ds(w * NCHUNK, NCHUNK)], idx_s)
    pltpu.sync_copy(dst_hbm.at[pl.ds(w * NCHUNK, NCHUNK)], idx_d)
    pltpu.sync_copy(ew_hbm.at[pl.ds(w * NCHUNK, NCHUNK)], wv)

    plsc.subcore_barrier()

    def _scale(j, src, dst):
        @plsc.parallel_loop(0, CH // 16)
        def _grp(m):
            wvec = wv[j, pl.ds(16 * m, 16)]
            for g in range(16):
                e = 16 * m + g
                we = wvec[g]
                for k in range(H // 32):
                    x = src[e, pl.ds(32 * k, 32)]
                    a, bb = plsc.unpack(x, format=plsc.PackFormat.INTERLEAVED)
                    dst[e, pl.ds(32 * k, 16)] = a * we
                    dst[e, pl.ds(32 * k + 16, 16)] = bb * we

    def _fire_gather(j, b):
        pltpu.async_copy(p_hbm.at[idx_s.at[j]], rows[b], gsems[b])

    def _wait_gather(j, b):
        pltpu.make_async_copy(p_hbm.at[idx_s.at[j]], rows[b], gsems[b]).wait()

    def _fire_scatter(j, b):
        pltpu.async_copy(sbuf[b], q_sh.at[idx_d.at[j]], ssems[b], add=True)

    def _wait_scatter(j, b):
        pltpu.make_async_copy(sbuf[b], q_sh.at[idx_d.at[j]], ssems[b]).wait()

    for b in range(NB):
        _fire_gather(b, b)

    def _super(t, _):
        j0 = t * NB
        for b in range(NB):
            j = j0 + b
            _wait_gather(j, b)

            @pl.when(t > 0)
            def _():
                _wait_scatter(j - NB, b)

            _scale(j, rows[b], sbuf[b])
            _fire_scatter(j, b)

            @pl.when(j + NB < NCHUNK)
            def _():
                _fire_gather(j + NB, b)
        return 0

    lax.fori_loop(0, NCHUNK // NB, _super, 0)
    for b in range(NB):
        _wait_scatter(NCHUNK - NB + b, b)

    plsc.subcore_barrier()

    pltpu.sync_copy(q_sh.at[pl.ds(s * NPS, NPS)],
                    qout_hbm.at[c, pl.ds(s * NPS, NPS)])


@functools.partial(
    pl.kernel,
    out_type=jax.ShapeDtypeStruct((NC, NPAD, H), jnp.float32),
    mesh=_sc_mesh,
    scratch_types=(
        [
            pltpu.VMEM_SHARED((NPAD, H), jnp.float32),
            pltpu.VMEM((NCHUNK, CH), jnp.int32),
            pltpu.VMEM((NCHUNK, CH), jnp.int32),
            pltpu.VMEM((NCHUNK, CH), jnp.float32),
        ]
        + [pltpu.VMEM((CH, H), jnp.bfloat16)] * NB
        + [pltpu.VMEM((CH, H), jnp.float32)] * NB
        + [pltpu.SemaphoreType.DMA] * (2 * NB)
    ),
    compiler_params=_sc_params,
)
def _message_pass(p_hbm, src_hbm, dst_hbm, ew_hbm, qout_hbm,
                  q_sh, idx_s, idx_d, wv, *bufs):
    rows = list(bufs[0:NB])
    sbuf = list(bufs[NB : 2 * NB])
    gsems = list(bufs[2 * NB : 3 * NB])
    ssems = list(bufs[3 * NB : 4 * NB])
    _mp_body(p_hbm, src_hbm, dst_hbm, ew_hbm, qout_hbm,
             q_sh, idx_s, idx_d, wv, rows, sbuf, gsems, ssems)


def _tc_first_body(x_ref, w_ref, degp_ref, h2_ref, p_ref, dinv_ref):
    deg = degp_ref[0, 0:N] + degp_ref[1, 0:N] + 1.0
    dinv = lax.rsqrt(deg)[:, None]
    dinv_ref[...] = dinv
    h2 = jnp.dot(x_ref[...], w_ref[...], preferred_element_type=jnp.float32)
    h2_ref[...] = h2
    p_ref[...] = (h2 * dinv).astype(jnp.bfloat16)


def _tc_first(x, w0, degp):
    return pl.pallas_call(
        _tc_first_body,
        out_shape=(
            jax.ShapeDtypeStruct((N, H), jnp.float32),
            jax.ShapeDtypeStruct((N, H), jnp.bfloat16),
            jax.ShapeDtypeStruct((N, 1), jnp.float32),
        ),
    )(x, w0, degp)


def _tc_mid_body(q_ref, h2_ref, dinv_ref, bgbt_ref, perm_ref, wn_ref,
                 h2n_ref, pn_ref):
    qperm = q_ref[0, :N, :] + q_ref[1, :N, :]
    q = jnp.dot(qperm, perm_ref[...], preferred_element_type=jnp.float32)
    dinv = dinv_ref[...]
    b = bgbt_ref[0:1, :]
    g = bgbt_ref[1:2, :]
    bt = bgbt_ref[2:3, :]
    acc = dinv * q + (dinv * dinv) * h2_ref[...] + b
    h = jnp.maximum(acc * (SCALE * g) + bt, 0.0)
    h2n = jnp.dot(h, wn_ref[...], preferred_element_type=jnp.float32)
    h2n_ref[...] = h2n
    pn_ref[...] = (h2n * dinv).astype(jnp.bfloat16)


def _tc_mid(qp, h2, dinv, b, g, bt, perm, wn):
    bgbt = jnp.stack([b, g, bt], axis=0)
    return pl.pallas_call(
        _tc_mid_body,
        out_shape=(
            jax.ShapeDtypeStruct((N, H), jnp.float32),
            jax.ShapeDtypeStruct((N, H), jnp.bfloat16),
        ),
    )(qp, h2, dinv, bgbt, perm, wn)


def _tc_last_body(q_ref, h2_ref, dinv_ref, bgbt_ref, perm_ref, batch_ref,
                  out_ref):
    qperm = q_ref[0, :N, :] + q_ref[1, :N, :]
    q = jnp.dot(qperm, perm_ref[...], preferred_element_type=jnp.float32)
    dinv = dinv_ref[...]
    b = bgbt_ref[0:1, :]
    g = bgbt_ref[1:2, :]
    bt = bgbt_ref[2:3, :]
    acc = dinv * q + (dinv * dinv) * h2_ref[...] + b
    h = jnp.maximum(acc * (SCALE * g) + bt, 0.0)
    seg = batch_ref[...]
    ids = lax.broadcasted_iota(jnp.int32, (B, N), 0)
    onehot = jnp.where(seg == ids, 1.0, 0.0)
    sums = jnp.dot(onehot, h, preferred_element_type=jnp.float32)
    counts = jnp.sum(onehot, axis=1, keepdims=True)
    mean = sums / jnp.maximum(counts, 1.0)
    out_ref[0:B, :] = mean
    out_ref[B : 2 * B, :] = sums


def _tc_last(qp, h2, dinv, b, g, bt, perm, batch):
    bgbt = jnp.stack([b, g, bt], axis=0)
    return pl.pallas_call(
        _tc_last_body,
        out_shape=jax.ShapeDtypeStruct((2 * B, H), jnp.float32),
    )(qp, h2, dinv, bgbt, perm, batch.reshape(1, N).astype(jnp.int32))


def kernel(x, edge_index, edge_weight, batch, W0, b0, g0, bt0, W1, b1, g1, bt1, W2, b2, g2, bt2):
    src = edge_index[0].astype(jnp.int32)
    dst = edge_index[1].astype(jnp.int32)
    ew = jnp.abs(edge_weight)

    pad = EPAD - E
    src2d = jnp.concatenate([src, jnp.zeros((pad,), jnp.int32)]).reshape(-1, CH)
    dst2d = jnp.concatenate([dst, jnp.zeros((pad,), jnp.int32)]).reshape(-1, CH)
    ew2d = jnp.concatenate([ew, jnp.zeros((pad,), jnp.float32)]).reshape(-1, CH)

    perm = jnp.asarray(_PERM)

    degp = _degree(dst2d, ew2d)
    h2, p, dinv = _tc_first(x, W0, degp)
    qp = _message_pass(p, src2d, dst2d, ew2d)
    h2, p = _tc_mid(qp, h2, dinv, b0, g0, bt0, perm, W1)
    qp = _message_pass(p, src2d, dst2d, ew2d)
    h2, p = _tc_mid(qp, h2, dinv, b1, g1, bt1, perm, W2)
    qp = _message_pass(p, src2d, dst2d, ew2d)
    out = _tc_last(qp, h2, dinv, b2, g2, bt2, perm, batch)
    return jnp.concatenate([out[0:B], out[B : 2 * B]], axis=-1)
